# baseline (speedup 1.0000x reference)
import functools

import jax
import jax.numpy as jnp
from jax.experimental import pallas as pl
from jax.experimental.pallas import tpu as pltpu

_B, _N, _M = 8192, 4096, 4096

_WS = 512


def _build_body(wm_ref, eps_ref, wu_ref, wv_ref, wb_ref):
    su = jnp.exp(0.5 * wu_ref[...])
    sv = jnp.exp(0.5 * wv_ref[...])
    wb_ref[...] = (wm_ref[...] + su * (eps_ref[...] * sv)).astype(jnp.bfloat16)


def _build_w(W_m, eps, wu2, wv2, interpret):
    return pl.pallas_call(
        _build_body,
        grid=(_N // _WS,),
        in_specs=[
            pl.BlockSpec((_WS, _M), lambda s: (s, 0)),
            pl.BlockSpec((_WS, _M), lambda s: (s, 0)),
            pl.BlockSpec((_WS, 1), lambda s: (s, 0)),
            pl.BlockSpec((1, _M), lambda s: (0, 0)),
        ],
        out_specs=pl.BlockSpec((_WS, _M), lambda s: (s, 0)),
        out_shape=jax.ShapeDtypeStruct((_N, _M), jnp.bfloat16),
        compiler_params=pltpu.CompilerParams(
            dimension_semantics=("parallel",),
            vmem_limit_bytes=59904 * 1024,
        ),
        name="mvg_build_w",
        interpret=interpret,
    )(W_m, eps, wu2, wv2)


_BM = 512
_BN = 512
_NJ = _M // _BN
_NI = _B // _BM
_HALF = _NI // 2


def _gemm_body(x_ref, wb_hbm, o_ref, xb_ref, wb_vmem, sem):
    i = pl.program_id(0)
    j = pl.program_id(1)
    first_i = jnp.logical_or(i == 0, i == _HALF)

    @pl.when(jnp.logical_and(first_i, j == 0))
    def _load_w():
        for js in range(_NJ):
            pltpu.make_async_copy(
                wb_hbm.at[:, js * _BN:(js + 1) * _BN],
                wb_vmem.at[js], sem.at[js]).start()
        for js in range(_NJ):
            pltpu.make_async_copy(
                wb_hbm.at[:, js * _BN:(js + 1) * _BN],
                wb_vmem.at[js], sem.at[js]).wait()

    @pl.when(j == 0)
    def _cast_x():
        xb_ref[...] = x_ref[...].astype(jnp.bfloat16)

    o_ref[...] = jnp.dot(xb_ref[...], wb_vmem[j],
                         preferred_element_type=jnp.float32)


def _gemm(x, wb, interpret):
    return pl.pallas_call(
        _gemm_body,
        grid=(_NI, _NJ),
        in_specs=[
            pl.BlockSpec((_BM, _N), lambda i, j: (i, 0)),
            pl.BlockSpec(memory_space=pl.ANY),
        ],
        out_specs=pl.BlockSpec((_BM, _BN), lambda i, j: (i, j)),
        out_shape=jax.ShapeDtypeStruct((_B, _M), jnp.float32),
        scratch_shapes=[
            pltpu.VMEM((_BM, _N), jnp.bfloat16),
            pltpu.VMEM((_NJ, _N, _BN), jnp.bfloat16),
            pltpu.SemaphoreType.DMA((_NJ,)),
        ],
        compiler_params=pltpu.CompilerParams(
            dimension_semantics=("parallel", "arbitrary"),
            vmem_limit_bytes=59904 * 1024,
        ),
        name="mvg_gemm",
        interpret=interpret,
    )(x, wb)


@functools.partial(jax.jit, static_argnames=("interpret",))
def kernel(x, W_m, W_u, W_v, eps, interpret=False):
    wu2 = W_u.reshape(_N, 1)
    wv2 = W_v.reshape(1, _M)
    wb = _build_w(W_m, eps, wu2, wv2, interpret)
    return _gemm(x, wb, interpret)


# speedup vs baseline: 1.0951x; 1.0951x over previous
import functools

import jax
import jax.numpy as jnp
from jax.experimental import pallas as pl
from jax.experimental.pallas import tpu as pltpu

_B, _N, _M = 8192, 4096, 4096

_WS = 512


def _build_body(wm_ref, eps_ref, wu_ref, wv_ref, wb_ref):
    su = jnp.exp(0.5 * wu_ref[...])
    sv = jnp.exp(0.5 * wv_ref[...])
    wb_ref[...] = (wm_ref[...] + su * (eps_ref[...] * sv)).astype(jnp.bfloat16)


def _build_w(W_m, eps, wu2, wv2, interpret):
    return pl.pallas_call(
        _build_body,
        grid=(_N // _WS,),
        in_specs=[
            pl.BlockSpec((_WS, _M), lambda s: (s, 0)),
            pl.BlockSpec((_WS, _M), lambda s: (s, 0)),
            pl.BlockSpec((_WS, 1), lambda s: (s, 0)),
            pl.BlockSpec((1, _M), lambda s: (0, 0)),
        ],
        out_specs=pl.BlockSpec((_WS, _M), lambda s: (s, 0)),
        out_shape=jax.ShapeDtypeStruct((_N, _M), jnp.bfloat16),
        compiler_params=pltpu.CompilerParams(
            dimension_semantics=("parallel",),
            vmem_limit_bytes=59904 * 1024,
        ),
        name="mvg_build_w",
        interpret=interpret,
    )(W_m, eps, wu2, wv2)


_BM = 1024
_BN = 512
_NJ = _M // _BN
_NI = _B // _BM


def _gemm_body(x_ref, wb_ref, o_ref, xb_ref):
    j = pl.program_id(1)

    @pl.when(j == 0)
    def _cast_x():
        xb_ref[...] = x_ref[...].astype(jnp.bfloat16)

    o_ref[...] = jnp.dot(xb_ref[...], wb_ref[...],
                         preferred_element_type=jnp.float32)


def _gemm(x, wb, interpret):
    return pl.pallas_call(
        _gemm_body,
        grid=(_NI, _NJ),
        in_specs=[
            pl.BlockSpec((_BM, _N), lambda i, j: (i, 0)),
            pl.BlockSpec((_N, _BN), lambda i, j: (0, j)),
        ],
        out_specs=pl.BlockSpec((_BM, _BN), lambda i, j: (i, j)),
        out_shape=jax.ShapeDtypeStruct((_B, _M), jnp.float32),
        scratch_shapes=[
            pltpu.VMEM((_BM, _N), jnp.bfloat16),
        ],
        compiler_params=pltpu.CompilerParams(
            dimension_semantics=("parallel", "arbitrary"),
            vmem_limit_bytes=59904 * 1024,
        ),
        name="mvg_gemm",
        interpret=interpret,
    )(x, wb)


@functools.partial(jax.jit, static_argnames=("interpret",))
def kernel(x, W_m, W_u, W_v, eps, interpret=False):
    wu2 = W_u.reshape(_N, 1)
    wv2 = W_v.reshape(1, _M)
    wb = _build_w(W_m, eps, wu2, wv2, interpret)
    return _gemm(x, wb, interpret)


# speedup vs baseline: 1.1187x; 1.0215x over previous
import functools

import jax
import jax.numpy as jnp
from jax.experimental import pallas as pl
from jax.experimental.pallas import tpu as pltpu

_B, _N, _M = 8192, 4096, 4096

_WS = 512


def _build_body(wm_ref, eps_ref, wu_ref, wv_ref, wb_ref):
    su = jnp.exp(0.5 * wu_ref[...])
    sv = jnp.exp(0.5 * wv_ref[...])
    wb_ref[...] = (wm_ref[...] + su * (eps_ref[...] * sv)).astype(jnp.bfloat16)


def _build_w(W_m, eps, wu2, wv2, interpret):
    return pl.pallas_call(
        _build_body,
        grid=(_N // _WS,),
        in_specs=[
            pl.BlockSpec((_WS, _M), lambda s: (s, 0)),
            pl.BlockSpec((_WS, _M), lambda s: (s, 0)),
            pl.BlockSpec((_WS, 1), lambda s: (s, 0)),
            pl.BlockSpec((1, _M), lambda s: (0, 0)),
        ],
        out_specs=pl.BlockSpec((_WS, _M), lambda s: (s, 0)),
        out_shape=jax.ShapeDtypeStruct((_N, _M), jnp.bfloat16),
        compiler_params=pltpu.CompilerParams(
            dimension_semantics=("parallel",),
            vmem_limit_bytes=59904 * 1024,
        ),
        name="mvg_build_w",
        interpret=interpret,
    )(W_m, eps, wu2, wv2)


_BM = 1024
_BN = 512
_NJ = _M // _BN
_NI = _B // _BM


def _gemm_body(x_ref, wb_ref, o_ref):
    o_ref[...] = jnp.dot(x_ref[...], wb_ref[...],
                         preferred_element_type=jnp.float32)


def _gemm(x, wb, interpret):
    return pl.pallas_call(
        _gemm_body,
        grid=(_NI, _NJ),
        in_specs=[
            pl.BlockSpec((_BM, _N), lambda i, j: (i, 0)),
            pl.BlockSpec((_N, _BN), lambda i, j: (0, j)),
        ],
        out_specs=pl.BlockSpec((_BM, _BN), lambda i, j: (i, j)),
        out_shape=jax.ShapeDtypeStruct((_B, _M), jnp.float32),
        compiler_params=pltpu.CompilerParams(
            dimension_semantics=("parallel", "arbitrary"),
            vmem_limit_bytes=59904 * 1024,
        ),
        name="mvg_gemm",
        interpret=interpret,
    )(x, wb)


@functools.partial(jax.jit, static_argnames=("interpret",))
def kernel(x, W_m, W_u, W_v, eps, interpret=False):
    wu2 = W_u.reshape(_N, 1)
    wv2 = W_v.reshape(1, _M)
    wb = _build_w(W_m, eps, wu2, wv2, interpret)
    return _gemm(x, wb, interpret)
